# baseline (speedup 1.0000x reference)
import jax
import jax.numpy as jnp
from jax.experimental import pallas as pl

_B = 32
_HALF = 16
_P = 8732
_C = 21
_PBL = 640
_NBLK = 14
_EPS = 1e-7


def _loss_body(c_ref, q_ref, num_ref, cnt_ref):
    g = pl.program_id(0)

    @pl.when(g == 0)
    def _init():
        num_ref[...] = jnp.zeros_like(num_ref)
        cnt_ref[...] = jnp.zeros_like(cnt_ref)

    bg = c_ref[0]
    t = bg + _EPS
    kl_row = t * (jnp.log(t) - jnp.log(q_ref[0] + _EPS))
    cmax = c_ref[1]
    t = cmax + _EPS
    kl_row += t * (jnp.log(t) - jnp.log(q_ref[1] + _EPS))
    for cls in range(2, _C):
        v = c_ref[cls]
        cmax = jnp.maximum(cmax, v)
        t = v + _EPS
        kl_row += t * (jnp.log(t) - jnp.log(q_ref[cls] + _EPS))

    left = cmax > bg
    right = jnp.concatenate([left[_HALF:], left[:_HALF]], axis=0)
    lanes = jax.lax.broadcasted_iota(jnp.int32, (_B, _PBL), 1)
    valid = (g * _PBL + lanes) < _P
    m = jnp.logical_and(jnp.logical_and(left, jnp.logical_not(right)), valid)

    num_ref[...] += jnp.full((1, 1), jnp.sum(jnp.where(m, kl_row, 0.0)))
    cnt_ref[...] += jnp.full((1, 1), jnp.sum(jnp.where(m, 1.0, 0.0)))


def kernel(args, lam, conf, loc, conf_mix, loc_mix):
    del args, lam, loc, loc_mix
    conf_t = jnp.transpose(conf, (2, 0, 1))
    mix_t = jnp.transpose(conf_mix, (2, 0, 1))
    in_spec = pl.BlockSpec((_C, _B, _PBL), lambda g: (0, 0, g))
    out_spec = pl.BlockSpec((1, 1), lambda g: (0, 0))
    num, cnt = pl.pallas_call(
        _loss_body,
        grid=(_NBLK,),
        in_specs=[in_spec, in_spec],
        out_specs=[out_spec, out_spec],
        out_shape=[
            jax.ShapeDtypeStruct((1, 1), jnp.float32),
            jax.ShapeDtypeStruct((1, 1), jnp.float32),
        ],
    )(conf_t, mix_t)
    num = num[0, 0]
    cnt = cnt[0, 0]
    loss = jnp.where(cnt > 0, num / jnp.maximum(cnt, 1.0), jnp.float32(0.0))
    return (jnp.zeros((1,), dtype=jnp.float32), loss)


# speedup vs baseline: 1.0749x; 1.0749x over previous
import jax
import jax.numpy as jnp
from jax.experimental import pallas as pl

_B = 32
_HALF = 16
_BC = 8
_NBLK = _B // _BC
_P = 8732
_C = 21
_EPS = 1e-7


def _rows_body(c_ref, q_ref, kl_ref, left_ref):
    bg = c_ref[0]
    t = bg + _EPS
    kl_row = t * (jnp.log(t) - jnp.log(q_ref[0] + _EPS))
    cmax = c_ref[1]
    t = cmax + _EPS
    kl_row += t * (jnp.log(t) - jnp.log(q_ref[1] + _EPS))
    for cls in range(2, _C):
        v = c_ref[cls]
        cmax = jnp.maximum(cmax, v)
        t = v + _EPS
        kl_row += t * (jnp.log(t) - jnp.log(q_ref[cls] + _EPS))

    kl_ref[...] = kl_row
    left_ref[...] = jnp.where(cmax > bg, 1.0, 0.0)


def _reduce_body(kl_ref, left_ref, num_ref, cnt_ref):
    left = left_ref[...] > 0.5
    right = jnp.concatenate([left[_HALF:], left[:_HALF]], axis=0)
    m = jnp.logical_and(left, jnp.logical_not(right))
    num_ref[...] = jnp.full((1, 1), jnp.sum(jnp.where(m, kl_ref[...], 0.0)))
    cnt_ref[...] = jnp.full((1, 1), jnp.sum(jnp.where(m, 1.0, 0.0)))


def kernel(args, lam, conf, loc, conf_mix, loc_mix):
    del args, lam, loc, loc_mix
    conf_t = jnp.transpose(conf, (2, 0, 1))
    mix_t = jnp.transpose(conf_mix, (2, 0, 1))
    in_spec = pl.BlockSpec((_C, _BC, _P), lambda g: (0, g, 0))
    plane_spec = pl.BlockSpec((_BC, _P), lambda g: (g, 0))
    kl_row, left = pl.pallas_call(
        _rows_body,
        grid=(_NBLK,),
        in_specs=[in_spec, in_spec],
        out_specs=[plane_spec, plane_spec],
        out_shape=[
            jax.ShapeDtypeStruct((_B, _P), jnp.float32),
            jax.ShapeDtypeStruct((_B, _P), jnp.float32),
        ],
    )(conf_t, mix_t)
    num, cnt = pl.pallas_call(
        _reduce_body,
        in_specs=[pl.BlockSpec((_B, _P), lambda: (0, 0))] * 2,
        out_specs=[pl.BlockSpec((1, 1), lambda: (0, 0))] * 2,
        out_shape=[
            jax.ShapeDtypeStruct((1, 1), jnp.float32),
            jax.ShapeDtypeStruct((1, 1), jnp.float32),
        ],
    )(kl_row, left)
    num = num[0, 0]
    cnt = cnt[0, 0]
    loss = jnp.where(cnt > 0, num / jnp.maximum(cnt, 1.0), jnp.float32(0.0))
    return (jnp.zeros((1,), dtype=jnp.float32), loss)


# speedup vs baseline: 1.2119x; 1.1274x over previous
import jax
import jax.numpy as jnp
from jax.experimental import pallas as pl
from jax.experimental.pallas import tpu as pltpu

_B = 32
_HALF = 16
_BC = 8
_NBLK = _B // _BC
_P = 8732
_C = 21
_EPS = 1e-7


def _loss_body(c_ref, q_ref, num_ref, cnt_ref, kl_scr, left_scr):
    g = pl.program_id(0)

    bg = c_ref[0]
    t = bg + _EPS
    kl_row = t * (jnp.log(t) - jnp.log(q_ref[0] + _EPS))
    cmax = c_ref[1]
    t = cmax + _EPS
    kl_row += t * (jnp.log(t) - jnp.log(q_ref[1] + _EPS))
    for cls in range(2, _C):
        v = c_ref[cls]
        cmax = jnp.maximum(cmax, v)
        t = v + _EPS
        kl_row += t * (jnp.log(t) - jnp.log(q_ref[cls] + _EPS))

    kl_scr[pl.ds(g * _BC, _BC), :] = kl_row
    left_scr[pl.ds(g * _BC, _BC), :] = jnp.where(cmax > bg, 1.0, 0.0)

    @pl.when(g == _NBLK - 1)
    def _reduce():
        left = left_scr[...] > 0.5
        right = jnp.concatenate([left[_HALF:], left[:_HALF]], axis=0)
        m = jnp.logical_and(left, jnp.logical_not(right))
        num_ref[...] = jnp.full((1, 1), jnp.sum(jnp.where(m, kl_scr[...], 0.0)))
        cnt_ref[...] = jnp.full((1, 1), jnp.sum(jnp.where(m, 1.0, 0.0)))


def kernel(args, lam, conf, loc, conf_mix, loc_mix):
    del args, lam, loc, loc_mix
    conf_t = jnp.transpose(conf, (2, 0, 1))
    mix_t = jnp.transpose(conf_mix, (2, 0, 1))
    in_spec = pl.BlockSpec((_C, _BC, _P), lambda g: (0, g, 0))
    out_spec = pl.BlockSpec((1, 1), lambda g: (0, 0))
    num, cnt = pl.pallas_call(
        _loss_body,
        grid=(_NBLK,),
        in_specs=[in_spec, in_spec],
        out_specs=[out_spec, out_spec],
        out_shape=[
            jax.ShapeDtypeStruct((1, 1), jnp.float32),
            jax.ShapeDtypeStruct((1, 1), jnp.float32),
        ],
        scratch_shapes=[
            pltpu.VMEM((_B, _P), jnp.float32),
            pltpu.VMEM((_B, _P), jnp.float32),
        ],
    )(conf_t, mix_t)
    num = num[0, 0]
    cnt = cnt[0, 0]
    loss = jnp.where(cnt > 0, num / jnp.maximum(cnt, 1.0), jnp.float32(0.0))
    return (jnp.zeros((1,), dtype=jnp.float32), loss)
